# Initial kernel scaffold; baseline (speedup 1.0000x reference)
import jax
import jax.numpy as jnp
from jax.experimental import pallas as pl


def kernel(x, edge_index, W_conv, b_conv, W1, b1, W2, b2, W3, b3, deterministic):
    raise NotImplementedError("write your pallas kernel here")


# speedup vs baseline: 11.3931x; 11.3931x over previous
import functools

import jax
import jax.numpy as jnp
from jax import lax
from jax.experimental import pallas as pl
from jax.experimental.pallas import tpu as pltpu
from jax.experimental.pallas import tpu_sc as plsc

N = 10000
D = 128
H = 256
E = 320000

NC = 2
NS = 16
NW = NC * NS

NP = 10240
RPT = NP // NS
BIN = N
CHUNK = 128
NCHUNK = -(-E // (NW * CHUNK))
EP = NW * NCHUNK * CHUNK

B_TC = 2560
GRID_TC = NP // B_TC

@functools.cache
def _sc_kernels():
    mesh = plsc.VectorSubcoreMesh(
        core_axis_name="c", subcore_axis_name="s",
        num_cores=NC, num_subcores=NS)

    @functools.partial(
        pl.kernel,
        out_type=jax.ShapeDtypeStruct((NC, NP, D), jnp.float32),
        mesh=mesh,
        scratch_types=[
            pltpu.VMEM((CHUNK,), jnp.int32),
            pltpu.VMEM((CHUNK, D), jnp.float32),
            pltpu.VMEM_SHARED((NP, D), jnp.float32),
            pltpu.SemaphoreType.DMA,
        ],
    )
    def _deg_kernel(dst_hbm, ones_hbm, zeros_hbm, out_hbm,
                    idx_v, ones_v, deg_sh, sem):
        c = lax.axis_index("c")
        s = lax.axis_index("s")
        wid = s * NC + c
        r0 = pl.multiple_of(s * RPT, 8)
        pltpu.sync_copy(zeros_hbm.at[pl.ds(r0, RPT)], deg_sh.at[pl.ds(r0, RPT)])
        pltpu.sync_copy(ones_hbm, ones_v)
        plsc.subcore_barrier()

        def body(j, carry):
            base = pl.multiple_of((wid * NCHUNK + j) * CHUNK, 128)
            pltpu.sync_copy(dst_hbm.at[pl.ds(base, CHUNK)], idx_v)
            pltpu.sync_copy(ones_v, deg_sh.at[idx_v], add=True)
            return carry

        lax.fori_loop(0, NCHUNK, body, 0)
        plsc.subcore_barrier()
        pltpu.sync_copy(deg_sh.at[pl.ds(r0, RPT)], out_hbm.at[c, pl.ds(r0, RPT)])

    @functools.partial(
        pl.kernel,
        out_type=jax.ShapeDtypeStruct((NC, NP, D), jnp.float32),
        mesh=mesh,
        scratch_types=[
            pltpu.VMEM((CHUNK,), jnp.int32),
            pltpu.VMEM((CHUNK,), jnp.int32),
            pltpu.VMEM((CHUNK, D), jnp.float32),
            pltpu.VMEM_SHARED((NP, D), jnp.float32),
            pltpu.SemaphoreType.DMA,
        ],
    )
    def _scat_kernel(hp_hbm, src_hbm, dst_hbm, zeros_hbm, out_hbm,
                     si_v, di_v, rows_v, acc_sh, sem):
        c = lax.axis_index("c")
        s = lax.axis_index("s")
        wid = s * NC + c
        r0 = pl.multiple_of(s * RPT, 8)
        pltpu.sync_copy(zeros_hbm.at[pl.ds(r0, RPT)], acc_sh.at[pl.ds(r0, RPT)])
        plsc.subcore_barrier()

        def body(j, carry):
            base = pl.multiple_of((wid * NCHUNK + j) * CHUNK, 128)
            pltpu.sync_copy(src_hbm.at[pl.ds(base, CHUNK)], si_v)
            pltpu.sync_copy(dst_hbm.at[pl.ds(base, CHUNK)], di_v)
            pltpu.async_copy(hp_hbm.at[si_v], rows_v, sem).wait()
            pltpu.sync_copy(rows_v, acc_sh.at[di_v], add=True)
            return carry

        lax.fori_loop(0, NCHUNK, body, 0)
        plsc.subcore_barrier()
        pltpu.sync_copy(acc_sh.at[pl.ds(r0, RPT)], out_hbm.at[c, pl.ds(r0, RPT)])

    return _deg_kernel, _scat_kernel


def _h_body(x_ref, w_ref, deg_ref, hp_ref, dinv_ref):
    i = pl.program_id(0)
    deg = deg_ref[0, :, 0] + deg_ref[1, :, 0] + 1.0
    dinv = lax.rsqrt(deg)
    h = jnp.dot(x_ref[...], w_ref[...], preferred_element_type=jnp.float32)
    rows = i * B_TC + lax.broadcasted_iota(jnp.int32, (B_TC, 1), 0)
    hp_ref[...] = jnp.where(rows < N, h * dinv[:, None], 0.0)
    dinv_ref[...] = jnp.broadcast_to(dinv[:, None], (B_TC, 8))


def _h_call(x_p, w, deg8):
    return pl.pallas_call(
        _h_body,
        grid=(GRID_TC,),
        in_specs=[
            pl.BlockSpec((B_TC, D), lambda i: (i, 0)),
            pl.BlockSpec((D, D), lambda i: (0, 0)),
            pl.BlockSpec((NC, B_TC, D), lambda i: (0, i, 0)),
        ],
        out_specs=[
            pl.BlockSpec((B_TC, D), lambda i: (i, 0)),
            pl.BlockSpec((B_TC, 8), lambda i: (i, 0)),
        ],
        out_shape=[
            jax.ShapeDtypeStruct((NP, D), jnp.float32),
            jax.ShapeDtypeStruct((NP, 8), jnp.float32),
        ],
    )(x_p, w, deg8)


def _lrelu(v):
    return jnp.where(v >= 0, v, 0.01 * v)


def _softplus(v):
    return jnp.maximum(v, 0.0) + jnp.log1p(jnp.exp(-jnp.abs(v)))


def _mlp_body(acc_ref, hp_ref, dinv_ref, x_ref, bconv_ref, w1_ref, b1_ref,
              w2_ref, b2_ref, w3_ref, b3_ref, shift_ref, conc_ref, s_ref):
    i = pl.program_id(0)
    dinv = dinv_ref[:, 0:1]
    g = (acc_ref[0] + acc_ref[1] + hp_ref[...]) * dinv + bconv_ref[...]
    g = jnp.maximum(g, 0.0) + x_ref[...]
    h1 = _lrelu(jnp.dot(g, w1_ref[...], preferred_element_type=jnp.float32)
                + b1_ref[...])
    h2 = _lrelu(jnp.dot(h1, w2_ref[...], preferred_element_type=jnp.float32)
                + b2_ref[...])
    h3 = _softplus(jnp.dot(h2, w3_ref[...], preferred_element_type=jnp.float32)
                   + b3_ref[...])
    conc = h3 + shift_ref[...]
    conc_ref[...] = conc
    rows = i * B_TC + lax.broadcasted_iota(jnp.int32, (B_TC, 1), 0)
    c2 = jnp.where(rows[:, 0] < N, conc[:, 2], 0.0)

    @pl.when(i == 0)
    def _():
        s_ref[0, 0] = 0.0

    s_ref[0, 0] += jnp.sum(c2)


def _mlp_call(acc, hp, dinv8, x_p, bconv, w1, b1, w2, b2, w3p, b3p, shift):
    return pl.pallas_call(
        _mlp_body,
        grid=(GRID_TC,),
        in_specs=[
            pl.BlockSpec((NC, B_TC, D), lambda i: (0, i, 0)),
            pl.BlockSpec((B_TC, D), lambda i: (i, 0)),
            pl.BlockSpec((B_TC, 8), lambda i: (i, 0)),
            pl.BlockSpec((B_TC, D), lambda i: (i, 0)),
            pl.BlockSpec((1, D), lambda i: (0, 0)),
            pl.BlockSpec((D, H), lambda i: (0, 0)),
            pl.BlockSpec((1, H), lambda i: (0, 0)),
            pl.BlockSpec((H, H), lambda i: (0, 0)),
            pl.BlockSpec((1, H), lambda i: (0, 0)),
            pl.BlockSpec((H, D), lambda i: (0, 0)),
            pl.BlockSpec((1, D), lambda i: (0, 0)),
            pl.BlockSpec((1, D), lambda i: (0, 0)),
        ],
        out_specs=[
            pl.BlockSpec((B_TC, D), lambda i: (i, 0)),
            pl.BlockSpec((1, 1), lambda i: (0, 0), memory_space=pltpu.SMEM),
        ],
        out_shape=[
            jax.ShapeDtypeStruct((NP, D), jnp.float32),
            jax.ShapeDtypeStruct((1, 1), jnp.float32),
        ],
    )(acc, hp, dinv8, x_p, bconv, w1, b1, w2, b2, w3p, b3p, shift)


def _act_body(conc_ref, s_ref, out_ref):
    c0 = conc_ref[:, 0]
    c1 = conc_ref[:, 1]
    c2 = conc_ref[:, 2]
    ao = c0 / (c0 + c1)
    ao = jnp.where(ao < 0.0, 0.0, ao)
    ar = c2 / s_ref[0, 0]
    lane = lax.broadcasted_iota(jnp.int32, (B_TC, 8), 1)
    out_ref[...] = jnp.where(lane == 0, ao[:, None],
                             jnp.where(lane == 1, ar[:, None], 0.0))


def _act_call(conc_t, s):
    return pl.pallas_call(
        _act_body,
        grid=(GRID_TC,),
        in_specs=[
            pl.BlockSpec((B_TC, D), lambda i: (i, 0)),
            pl.BlockSpec((1, 1), lambda i: (0, 0), memory_space=pltpu.SMEM),
        ],
        out_specs=pl.BlockSpec((B_TC, 8), lambda i: (i, 0)),
        out_shape=jax.ShapeDtypeStruct((NP, 8), jnp.float32),
    )(conc_t, s)


def kernel(x, edge_index, W_conv, b_conv, W1, b1, W2, b2, W3, b3,
           deterministic=True):
    src = edge_index[0]
    dst = edge_index[1]
    padv = jnp.full((EP - E,), BIN, dtype=jnp.int32)
    src_p = jnp.concatenate([src, padv])
    dst_p = jnp.concatenate([dst, padv])
    x_p = jnp.pad(x, ((0, NP - N), (0, 0)))
    ones8 = jnp.ones((CHUNK, D), jnp.float32)
    zerosD = jnp.zeros((NP, D), jnp.float32)
    w3p = jnp.pad(W3, ((0, 0), (0, D - 3)))
    b3p = jnp.pad(b3, (0, D - 3)).reshape(1, D)
    shift = jnp.zeros((1, D), jnp.float32).at[0, 0].set(1.0).at[0, 1].set(1.0) \
        .at[0, 2].set(0.1)

    deg_k, scat_k = _sc_kernels()
    deg8 = deg_k(dst_p, ones8, zerosD)
    hp, dinv8 = _h_call(x_p, W_conv, deg8)
    acc = scat_k(hp, src_p, dst_p, zerosD)
    conc_t, s = _mlp_call(acc, hp, dinv8, x_p, b_conv.reshape(1, D),
                          W1, b1.reshape(1, H), W2, b2.reshape(1, H),
                          w3p, b3p, shift)
    act = _act_call(conc_t, s)

    conc = conc_t[:N, :3].reshape(1, N, 3)
    action = act[:N, :2]
    return (action, conc)
